# Initial kernel scaffold; baseline (speedup 1.0000x reference)
import jax
import jax.numpy as jnp
from jax.experimental import pallas as pl


def kernel(atom_indices, valence_configs, core_configs, W_valence, W_core):
    raise NotImplementedError("write your pallas kernel here")


# speedup vs baseline: 2.3780x; 2.3780x over previous
import functools

import jax
import jax.numpy as jnp
from jax import lax
from jax.experimental import pallas as pl
from jax.experimental.pallas import tpu as pltpu
from jax.experimental.pallas import tpu_sc as plsc

K = 12
D = 64
N_ROWS = 19
B_TOT = 1024 * 200

NC, NS = 2, 16
NW = NC * NS
PER_W = B_TOT // NW
G = 128
NG = PER_W // G
BATCH = 10
N_OUTER = NG // BATCH
CHUNK = BATCH * G


def _table_body(vc_ref, cc_ref, wv_ref, wc_ref, out_ref):
    zero = jnp.zeros((1, K), jnp.float32)
    aug_v = jnp.concatenate([zero, vc_ref[...]], axis=0)
    aug_c = jnp.concatenate([zero, cc_ref[...]], axis=0)
    tv = jnp.dot(aug_v, wv_ref[...], preferred_element_type=jnp.float32,
                 precision=jax.lax.Precision.HIGHEST)
    tc = jnp.dot(aug_c, wc_ref[...], preferred_element_type=jnp.float32,
                 precision=jax.lax.Precision.HIGHEST)
    out_ref[...] = jnp.concatenate([tv, tc], axis=-1)


def _build_table(vc, cc, wv, wc):
    return pl.pallas_call(
        _table_body,
        out_shape=jax.ShapeDtypeStruct((N_ROWS, D), jnp.float32),
    )(vc, cc, wv, wc)


def _sc_body(table_hbm, idx_hbm, out_hbm, idx_v, rows_v, sem):
    wid = lax.axis_index("s") * NC + lax.axis_index("c")
    pltpu.sync_copy(idx_hbm.at[wid], idx_v)
    base = wid * PER_W
    for o in range(N_OUTER):
        cps = [
            pltpu.async_copy(
                table_hbm.at[idx_v.at[o * BATCH + g]],
                rows_v.at[pl.ds(g * G, G)],
                sem,
            )
            for g in range(BATCH)
        ]
        for cp in cps:
            cp.wait()
        pltpu.sync_copy(rows_v, out_hbm.at[pl.ds(base + o * CHUNK, CHUNK)])


_gather_rows = functools.partial(
    pl.kernel,
    out_type=jax.ShapeDtypeStruct((B_TOT, D), jnp.float32),
    mesh=plsc.VectorSubcoreMesh(core_axis_name="c", subcore_axis_name="s"),
    scratch_types=[
        pltpu.VMEM((NG, G), jnp.int32),
        pltpu.VMEM((CHUNK, D), jnp.float32),
        pltpu.SemaphoreType.DMA,
    ],
    compiler_params=pltpu.CompilerParams(use_tc_tiling_on_sc=False),
)(_sc_body)


def kernel(atom_indices, valence_configs, core_configs, W_valence, W_core):
    table = _build_table(valence_configs, core_configs, W_valence, W_core)
    idx = atom_indices.astype(jnp.int32).reshape(NW, NG, G)
    out = _gather_rows(table, idx)
    return out.reshape(atom_indices.shape[0], atom_indices.shape[1], D)


# speedup vs baseline: 2.4199x; 1.0176x over previous
import functools

import jax
import jax.numpy as jnp
from jax import lax
from jax.experimental import pallas as pl
from jax.experimental.pallas import tpu as pltpu
from jax.experimental.pallas import tpu_sc as plsc

K = 12
D = 64
N_ROWS = 19
B_TOT = 1024 * 200

NC, NS = 2, 16
NW = NC * NS
PER_W = B_TOT // NW
CHUNK = 800
N_CHUNK = PER_W // CHUNK
GROUPS = CHUNK // 16


def _table_body(vc_ref, cc_ref, wv_ref, wc_ref, out_ref):
    zero = jnp.zeros((1, K), jnp.float32)
    aug_v = jnp.concatenate([zero, vc_ref[...]], axis=0)
    aug_c = jnp.concatenate([zero, cc_ref[...]], axis=0)
    tv = jnp.dot(aug_v, wv_ref[...], preferred_element_type=jnp.float32,
                 precision=jax.lax.Precision.HIGHEST)
    tc = jnp.dot(aug_c, wc_ref[...], preferred_element_type=jnp.float32,
                 precision=jax.lax.Precision.HIGHEST)
    out_ref[...] = jnp.concatenate([tv, tc], axis=-1)


def _build_table(vc, cc, wv, wc):
    return pl.pallas_call(
        _table_body,
        out_shape=jax.ShapeDtypeStruct((N_ROWS, D), jnp.float32),
    )(vc, cc, wv, wc)


def _sc_body(table_hbm, idx_hbm, out_hbm, table_v, idx_v, stag_a, stag_b, sem_a, sem_b):
    wid = lax.axis_index("s") * NC + lax.axis_index("c")
    base = wid * PER_W
    pltpu.sync_copy(table_hbm, table_v)
    pltpu.sync_copy(idx_hbm.at[pl.ds(base, PER_W)], idx_v)

    iota64 = lax.iota(jnp.int32, 16) * D
    stags = (stag_a, stag_b)
    sems = (sem_a, sem_b)
    pending = [None, None]

    for k in range(N_CHUNK):
        b = k % 2
        if pending[b] is not None:
            pending[b].wait()
        stag = stags[b]

        def group_body(g, carry, _k=k, _stag=stag):
            v = idx_v[pl.ds(_k * CHUNK + g * 16, 16)]
            v64 = v * D
            stb = iota64 + g * (16 * D)
            for c in range(D):
                vals = plsc.load_gather(table_v, [v64 + c])
                plsc.store_scatter(_stag, [stb + c], vals)
            return carry

        lax.fori_loop(0, GROUPS, group_body, 0, unroll=False)
        pending[b] = pltpu.async_copy(
            stag,
            out_hbm.at[pl.ds((base + k * CHUNK) * D, CHUNK * D)],
            sems[b],
        )

    for b in range(2):
        if pending[b] is not None:
            pending[b].wait()


_gather_rows = functools.partial(
    pl.kernel,
    out_type=jax.ShapeDtypeStruct((B_TOT * D,), jnp.float32),
    mesh=plsc.VectorSubcoreMesh(core_axis_name="c", subcore_axis_name="s"),
    scratch_types=[
        pltpu.VMEM((N_ROWS * D,), jnp.float32),
        pltpu.VMEM((PER_W,), jnp.int32),
        pltpu.VMEM((CHUNK * D,), jnp.float32),
        pltpu.VMEM((CHUNK * D,), jnp.float32),
        pltpu.SemaphoreType.DMA,
        pltpu.SemaphoreType.DMA,
    ],
    compiler_params=pltpu.CompilerParams(
        use_tc_tiling_on_sc=False, needs_layout_passes=False
    ),
)(_sc_body)


def kernel(atom_indices, valence_configs, core_configs, W_valence, W_core):
    table = _build_table(valence_configs, core_configs, W_valence, W_core)
    idx = atom_indices.astype(jnp.int32).reshape(B_TOT)
    out = _gather_rows(table.reshape(N_ROWS * D), idx)
    return out.reshape(atom_indices.shape[0], atom_indices.shape[1], D)


# speedup vs baseline: 5.1341x; 2.1216x over previous
import functools

import jax
import jax.numpy as jnp
from jax import lax
from jax.experimental import pallas as pl
from jax.experimental.pallas import tpu as pltpu
from jax.experimental.pallas import tpu_sc as plsc

K = 12
D = 64
N_ROWS = 19
B_TOT = 1024 * 200

NC, NS = 2, 16
NW = NC * NS
PER_W = B_TOT // NW
CHUNK = 800
N_CHUNK = PER_W // CHUNK
GROUPS = CHUNK // 16


def _table_body(vc_ref, cc_ref, wv_ref, wc_ref, out_ref):
    zero = jnp.zeros((1, K), jnp.float32)
    aug_v = jnp.concatenate([zero, vc_ref[...]], axis=0)
    aug_c = jnp.concatenate([zero, cc_ref[...]], axis=0)
    tv = jnp.dot(aug_v, wv_ref[...], preferred_element_type=jnp.float32,
                 precision=jax.lax.Precision.HIGHEST)
    tc = jnp.dot(aug_c, wc_ref[...], preferred_element_type=jnp.float32,
                 precision=jax.lax.Precision.HIGHEST)
    out_ref[...] = jnp.concatenate([tv, tc], axis=-1)


def _build_table(vc, cc, wv, wc):
    return pl.pallas_call(
        _table_body,
        out_shape=jax.ShapeDtypeStruct((N_ROWS, D), jnp.float32),
    )(vc, cc, wv, wc)


def _sc_body(table_hbm, idx_hbm, out_hbm, table_v, idx_v, stag_a, stag_b, sem_a, sem_b):
    wid = lax.axis_index("s") * NC + lax.axis_index("c")
    base = wid * PER_W
    pltpu.sync_copy(table_hbm, table_v)
    pltpu.sync_copy(idx_hbm.at[pl.ds(base, PER_W)], idx_v)

    iota = lax.iota(jnp.int32, 16)
    iota64 = iota * D
    diags = [(iota + d) & 15 for d in range(16)]
    stags = (stag_a, stag_b)
    sems = (sem_a, sem_b)
    pending = [None, None]

    for k in range(N_CHUNK):
        b = k % 2
        if pending[b] is not None:
            pending[b].wait()
        stag = stags[b]

        def group_body(g, carry, _k=k, _stag=stag):
            v = idx_v[pl.ds(_k * CHUNK + g * 16, 16)]
            v64 = v * D
            sb = iota64 + g * (16 * D)
            for c0 in range(0, D, 16):
                pb = v64 + c0
                sbc = sb + c0
                for d in range(16):
                    vals = plsc.load_gather(table_v, [pb + diags[d]])
                    plsc.store_scatter(_stag, [sbc + diags[d]], vals)
            return carry

        lax.fori_loop(0, GROUPS, group_body, 0, unroll=False)
        pending[b] = pltpu.async_copy(
            stag,
            out_hbm.at[pl.ds((base + k * CHUNK) * D, CHUNK * D)],
            sems[b],
        )

    for b in range(2):
        if pending[b] is not None:
            pending[b].wait()


_gather_rows = functools.partial(
    pl.kernel,
    out_type=jax.ShapeDtypeStruct((B_TOT * D,), jnp.float32),
    mesh=plsc.VectorSubcoreMesh(core_axis_name="c", subcore_axis_name="s"),
    scratch_types=[
        pltpu.VMEM((N_ROWS * D,), jnp.float32),
        pltpu.VMEM((PER_W,), jnp.int32),
        pltpu.VMEM((CHUNK * D,), jnp.float32),
        pltpu.VMEM((CHUNK * D,), jnp.float32),
        pltpu.SemaphoreType.DMA,
        pltpu.SemaphoreType.DMA,
    ],
    compiler_params=pltpu.CompilerParams(
        use_tc_tiling_on_sc=False, needs_layout_passes=False
    ),
)(_sc_body)


def kernel(atom_indices, valence_configs, core_configs, W_valence, W_core):
    table = _build_table(valence_configs, core_configs, W_valence, W_core)
    idx = atom_indices.astype(jnp.int32).reshape(B_TOT)
    out = _gather_rows(table.reshape(N_ROWS * D), idx)
    return out.reshape(atom_indices.shape[0], atom_indices.shape[1], D)


# speedup vs baseline: 8.4181x; 1.6396x over previous
import functools

import jax
import jax.numpy as jnp
from jax import lax
from jax.experimental import pallas as pl
from jax.experimental.pallas import tpu as pltpu
from jax.experimental.pallas import tpu_sc as plsc

K = 12
D = 64
N_ROWS = 19
B_TOT = 1024 * 200

NC, NS = 2, 16
NW = NC * NS
PER_W = B_TOT // NW
CHUNK = 800
N_CHUNK = PER_W // CHUNK
GROUPS = CHUNK // 16


def _table_body(vc_ref, cc_ref, wv_ref, wc_ref, out_ref):
    zero = jnp.zeros((1, K), jnp.float32)
    aug_v = jnp.concatenate([zero, vc_ref[...]], axis=0)
    aug_c = jnp.concatenate([zero, cc_ref[...]], axis=0)
    tv = jnp.dot(aug_v, wv_ref[...], preferred_element_type=jnp.float32,
                 precision=jax.lax.Precision.HIGHEST)
    tc = jnp.dot(aug_c, wc_ref[...], preferred_element_type=jnp.float32,
                 precision=jax.lax.Precision.HIGHEST)
    out_ref[...] = jnp.concatenate([tv, tc], axis=-1)


def _build_table(vc, cc, wv, wc):
    return pl.pallas_call(
        _table_body,
        out_shape=jax.ShapeDtypeStruct((N_ROWS, D), jnp.float32),
    )(vc, cc, wv, wc)


def _sc_body(table_hbm, idx_hbm, out_hbm, table_v, idx_v, stag_a, stag_b, sem_a, sem_b):
    wid = lax.axis_index("s") * NC + lax.axis_index("c")
    base = wid * PER_W
    pltpu.sync_copy(table_hbm, table_v)
    pltpu.sync_copy(idx_hbm.at[pl.ds(base, PER_W)], idx_v)

    iota = lax.iota(jnp.int32, 16)
    iota64 = iota * D
    stags = (stag_a, stag_b)
    sems = (sem_a, sem_b)
    pending = [None, None]

    for k in range(N_CHUNK):
        b = k % 2
        if pending[b] is not None:
            pending[b].wait()
        stag = stags[b]

        @plsc.parallel_loop(0, GROUPS)
        def _group_body(g, _k=k, _stag=stag):
            v = idx_v[pl.ds(_k * CHUNK + g * 16, 16)]
            v64 = v * D
            goff = g * (16 * D)
            dg = iota
            for _d in range(16):
                pv = v64 + dg
                sv = iota64 + dg
                for c0 in range(0, D, 16):
                    vals = plsc.load_gather(
                        table_v.at[pl.ds(c0, (N_ROWS - 1) * D + 16)], [pv]
                    )
                    plsc.store_scatter(
                        _stag.at[pl.ds(goff + c0, 15 * D + 16)], [sv], vals
                    )
                dg = (dg + 1) & 15
        pending[b] = pltpu.async_copy(
            stag,
            out_hbm.at[pl.ds((base + k * CHUNK) * D, CHUNK * D)],
            sems[b],
        )

    for b in range(2):
        if pending[b] is not None:
            pending[b].wait()


_gather_rows = functools.partial(
    pl.kernel,
    out_type=jax.ShapeDtypeStruct((B_TOT * D,), jnp.float32),
    mesh=plsc.VectorSubcoreMesh(core_axis_name="c", subcore_axis_name="s"),
    scratch_types=[
        pltpu.VMEM((N_ROWS * D,), jnp.float32),
        pltpu.VMEM((PER_W,), jnp.int32),
        pltpu.VMEM((CHUNK * D,), jnp.float32),
        pltpu.VMEM((CHUNK * D,), jnp.float32),
        pltpu.SemaphoreType.DMA,
        pltpu.SemaphoreType.DMA,
    ],
    compiler_params=pltpu.CompilerParams(
        use_tc_tiling_on_sc=False, needs_layout_passes=False
    ),
)(_sc_body)


def kernel(atom_indices, valence_configs, core_configs, W_valence, W_core):
    table = _build_table(valence_configs, core_configs, W_valence, W_core)
    idx = jnp.minimum(atom_indices.astype(jnp.int32), N_ROWS - 1).reshape(B_TOT)
    out = _gather_rows(table.reshape(N_ROWS * D), idx)
    return out.reshape(atom_indices.shape[0], atom_indices.shape[1], D)


# speedup vs baseline: 8.4278x; 1.0012x over previous
import functools

import jax
import jax.numpy as jnp
from jax import lax
from jax.experimental import pallas as pl
from jax.experimental.pallas import tpu as pltpu
from jax.experimental.pallas import tpu_sc as plsc

K = 12
D = 64
N_ROWS = 19
B_TOT = 1024 * 200

NC, NS = 2, 16
NW = NC * NS
PER_W = B_TOT // NW
CHUNK = 800
N_CHUNK = PER_W // CHUNK
GROUPS = CHUNK // 16


def _table_body(vc_ref, cc_ref, wv_ref, wc_ref, out_ref):
    zero = jnp.zeros((1, K), jnp.float32)
    aug_v = jnp.concatenate([zero, vc_ref[...]], axis=0)
    aug_c = jnp.concatenate([zero, cc_ref[...]], axis=0)
    tv = jnp.dot(aug_v, wv_ref[...], preferred_element_type=jnp.float32,
                 precision=jax.lax.Precision.HIGHEST)
    tc = jnp.dot(aug_c, wc_ref[...], preferred_element_type=jnp.float32,
                 precision=jax.lax.Precision.HIGHEST)
    out_ref[...] = jnp.concatenate([tv, tc], axis=-1)


def _build_table(vc, cc, wv, wc):
    return pl.pallas_call(
        _table_body,
        out_shape=jax.ShapeDtypeStruct((N_ROWS, D), jnp.float32),
    )(vc, cc, wv, wc)


def _sc_body(table_hbm, idx_hbm, out_hbm, table_v, idx_v, stag_a, stag_b, sem_a, sem_b):
    wid = lax.axis_index("s") * NC + lax.axis_index("c")
    base = wid * PER_W
    pltpu.sync_copy(table_hbm, table_v)
    pltpu.sync_copy(idx_hbm.at[pl.ds(base, PER_W)], idx_v)

    iota = lax.iota(jnp.int32, 16)
    iota64 = iota * D
    stags = (stag_a, stag_b)
    sems = (sem_a, sem_b)
    pending = [None, None]

    for k in range(N_CHUNK):
        b = k % 2
        if pending[b] is not None:
            pending[b].wait()
        stag = stags[b]

        @plsc.parallel_loop(0, GROUPS)
        def _group_body(g, _k=k, _stag=stag):
            v = idx_v[pl.ds(_k * CHUNK + g * 16, 16)]
            v64 = v * D
            goff = g * (16 * D)
            dg = iota
            for _d in range(16):
                pv = v64 + dg
                sv = iota64 + dg
                for c0 in range(0, D, 16):
                    vals = plsc.load_gather(
                        table_v.at[pl.ds(c0, (N_ROWS - 1) * D + 16)], [pv]
                    )
                    plsc.store_scatter(
                        _stag.at[pl.ds(goff + c0, 15 * D + 16)], [sv], vals
                    )
                dg = (dg + 1) & 15
        pending[b] = pltpu.async_copy(
            stag,
            out_hbm.at[pl.ds((base + k * CHUNK) * D, CHUNK * D)],
            sems[b],
        )

    for b in range(2):
        if pending[b] is not None:
            pending[b].wait()


_gather_rows = functools.partial(
    pl.kernel,
    out_type=jax.ShapeDtypeStruct((B_TOT * D,), jnp.float32),
    mesh=plsc.VectorSubcoreMesh(core_axis_name="c", subcore_axis_name="s"),
    scratch_types=[
        pltpu.VMEM((N_ROWS * D,), jnp.float32),
        pltpu.VMEM((PER_W,), jnp.int32),
        pltpu.VMEM((CHUNK * D,), jnp.float32),
        pltpu.VMEM((CHUNK * D,), jnp.float32),
        pltpu.SemaphoreType.DMA,
        pltpu.SemaphoreType.DMA,
    ],
    compiler_params=pltpu.CompilerParams(
        use_tc_tiling_on_sc=False, needs_layout_passes=False
    ),
)(_sc_body)


def kernel(atom_indices, valence_configs, core_configs, W_valence, W_core):
    table = _build_table(valence_configs, core_configs, W_valence, W_core)
    idx = jnp.minimum(atom_indices.astype(jnp.int32).reshape(B_TOT), N_ROWS - 1)
    out = _gather_rows(table.reshape(N_ROWS * D), idx)
    return out.reshape(atom_indices.shape[0], atom_indices.shape[1], D)


# speedup vs baseline: 22.8217x; 2.7079x over previous
import functools

import jax
import jax.numpy as jnp
from jax import lax
from jax.experimental import pallas as pl
from jax.experimental.pallas import tpu as pltpu
from jax.experimental.pallas import tpu_sc as plsc

K = 12
D = 64
N_ROWS = 19
B = 1024
L = 200

NC, NS = 2, 16
NW = NC * NS
L_UNITS_MAX = (L + NW - 1) // NW + 1
UNIT = 2 * 8 * 8 * 128
SLAB = D * B


def _table_body(vc_ref, cc_ref, wv_ref, wc_ref, out_ref):
    zero = jnp.zeros((1, K), jnp.float32)
    aug_v = jnp.concatenate([zero, vc_ref[...]], axis=0)
    aug_c = jnp.concatenate([zero, cc_ref[...]], axis=0)
    tv = jnp.dot(aug_v, wv_ref[...], preferred_element_type=jnp.float32,
                 precision=jax.lax.Precision.HIGHEST)
    tc = jnp.dot(aug_c, wc_ref[...], preferred_element_type=jnp.float32,
                 precision=jax.lax.Precision.HIGHEST)
    out_ref[...] = jnp.concatenate([tv, tc], axis=-1)


def _build_table(vc, cc, wv, wc):
    return pl.pallas_call(
        _table_body,
        out_shape=jax.ShapeDtypeStruct((N_ROWS, D), jnp.float32),
    )(vc, cc, wv, wc)


def _sc_body(table_hbm, idx_hbm, out_hbm,
             table_v, icol, svtab, stag_a, stag_b, sem_a, sem_b, isem):
    w = lax.axis_index("s") * NC + lax.axis_index("c")
    pltpu.sync_copy(table_hbm, table_v)

    iota = lax.iota(jnp.int32, 16)
    for d in range(16):
        cr = (iota + d) & 15
        svtab[pl.ds(d * 16, 16)] = (cr >> 3) * 8192 + (cr & 7) * 128 + iota

    lo = (w * L) // NW
    hi = ((w + 1) * L) // NW
    stags = (stag_a, stag_b)
    sems = (sem_a, sem_b)

    def _slab(l, carry):
        lt = l >> 3
        li = l & 7
        cps = [
            pltpu.async_copy(
                idx_hbm.at[lt, bb, li], icol.at[pl.ds(bb * 128, 128)], isem
            )
            for bb in range(8)
        ]
        for cp in cps:
            cp.wait()
        for h in range(2):
            stag = stags[h]
            sem = sems[h]

            @pl.when(l > lo)
            def _drain(_stag=stag, _sem=sem):
                pltpu.make_async_copy(out_hbm.at[0], _stag, _sem).wait()

            @plsc.parallel_loop(0, 128)
            def _group(g2, _stag=stag, _h=h):
                cqr = g2 >> 6
                g = g2 & 63
                v = icol[pl.ds(g * 16, 16)]
                v64 = v * D
                goff = cqr * UNIT + (g >> 3) * 1024 + (g & 7) * 16
                tb = _h * 32 + cqr * 16
                dg = iota
                for d in range(16):
                    vals = plsc.load_gather(
                        table_v.at[pl.ds(tb, (N_ROWS - 1) * D + 16)],
                        [v64 + dg],
                    )
                    sv = svtab[pl.ds(d * 16, 16)]
                    plsc.store_scatter(
                        _stag.at[pl.ds(goff, 8192 + 896 + 16)], [sv], vals
                    )
                    dg = (dg + 1) & 15

            pltpu.async_copy(stag, out_hbm.at[l * 2 + h], sem)
        return carry

    lax.fori_loop(lo, hi, _slab, 0)
    for h in range(2):
        pltpu.make_async_copy(out_hbm.at[0], stags[h], sems[h]).wait()


_gather_rows = functools.partial(
    pl.kernel,
    out_type=jax.ShapeDtypeStruct((L * 2, 2 * UNIT), jnp.float32),
    mesh=plsc.VectorSubcoreMesh(core_axis_name="c", subcore_axis_name="s"),
    scratch_types=[
        pltpu.VMEM((N_ROWS * D,), jnp.float32),
        pltpu.VMEM((1024,), jnp.int32),
        pltpu.VMEM((256,), jnp.int32),
        pltpu.VMEM((2 * UNIT,), jnp.float32),
        pltpu.VMEM((2 * UNIT,), jnp.float32),
        pltpu.SemaphoreType.DMA,
        pltpu.SemaphoreType.DMA,
        pltpu.SemaphoreType.DMA,
    ],
    compiler_params=pltpu.CompilerParams(
        use_tc_tiling_on_sc=False, needs_layout_passes=False
    ),
)(_sc_body)


def kernel(atom_indices, valence_configs, core_configs, W_valence, W_core):
    table = _build_table(valence_configs, core_configs, W_valence, W_core)
    idx4 = (
        atom_indices.astype(jnp.int32)
        .reshape(8, 128, L // 8, 8)
        .transpose(2, 0, 3, 1)
    )
    out = _gather_rows(table.reshape(N_ROWS * D), idx4)
    return (
        out.reshape(L, 8, 8, 8, 128)
        .transpose(2, 4, 0, 1, 3)
        .reshape(B, L, D)
    )


# speedup vs baseline: 22.9358x; 1.0050x over previous
import functools

import jax
import jax.numpy as jnp
from jax import lax
from jax.experimental import pallas as pl
from jax.experimental.pallas import tpu as pltpu
from jax.experimental.pallas import tpu_sc as plsc

K = 12
D = 64
N_ROWS = 19
B = 1024
L = 200

NC, NS = 2, 16
NW = NC * NS
L_UNITS_MAX = (L + NW - 1) // NW + 1
UNIT = 2 * 8 * 8 * 128
SLAB = D * B


def _table_body(vc_ref, cc_ref, wv_ref, wc_ref, out_ref):
    zero = jnp.zeros((1, K), jnp.float32)
    aug_v = jnp.concatenate([zero, vc_ref[...]], axis=0)
    aug_c = jnp.concatenate([zero, cc_ref[...]], axis=0)
    tv = jnp.dot(aug_v, wv_ref[...], preferred_element_type=jnp.float32,
                 precision=jax.lax.Precision.HIGHEST)
    tc = jnp.dot(aug_c, wc_ref[...], preferred_element_type=jnp.float32,
                 precision=jax.lax.Precision.HIGHEST)
    out_ref[...] = jnp.concatenate([tv, tc], axis=-1)


def _build_table(vc, cc, wv, wc):
    return pl.pallas_call(
        _table_body,
        out_shape=jax.ShapeDtypeStruct((N_ROWS, D), jnp.float32),
    )(vc, cc, wv, wc)


def _sc_body(table_hbm, idx_hbm, out_hbm,
             table_v, icol, svtab, stag_a, stag_b, sem_a, sem_b, isem):
    w = lax.axis_index("s") * NC + lax.axis_index("c")
    pltpu.sync_copy(table_hbm, table_v)

    iota = lax.iota(jnp.int32, 16)
    for d in range(16):
        cr = (iota + d) & 15
        svtab[pl.ds(d * 16, 16)] = (cr >> 3) * 8192 + (cr & 7) * 128 + iota

    lo = (w * L) // NW
    hi = ((w + 1) * L) // NW
    stags = (stag_a, stag_b)
    sems = (sem_a, sem_b)

    def _slab(l, carry):
        lt = l >> 3
        li = l & 7
        cps = [
            pltpu.async_copy(
                idx_hbm.at[lt, bb, li], icol.at[pl.ds(bb * 128, 128)], isem
            )
            for bb in range(8)
        ]
        for cp in cps:
            cp.wait()
        for h in range(2):
            stag = stags[h]
            sem = sems[h]

            @pl.when(l > lo)
            def _drain(_stag=stag, _sem=sem):
                pltpu.make_async_copy(out_hbm.at[0], _stag, _sem).wait()

            @plsc.parallel_loop(0, 128, unroll=2)
            def _group(g2, _stag=stag, _h=h):
                cqr = g2 >> 6
                g = g2 & 63
                v = icol[pl.ds(g * 16, 16)]
                v64 = v * D
                goff = cqr * UNIT + (g >> 3) * 1024 + (g & 7) * 16
                tb = _h * 32 + cqr * 16
                dg = iota
                for d in range(16):
                    vals = plsc.load_gather(
                        table_v.at[pl.ds(tb, (N_ROWS - 1) * D + 16)],
                        [v64 + dg],
                    )
                    sv = svtab[pl.ds(d * 16, 16)]
                    plsc.store_scatter(
                        _stag.at[pl.ds(goff, 8192 + 896 + 16)], [sv], vals
                    )
                    dg = (dg + 1) & 15

            pltpu.async_copy(stag, out_hbm.at[l * 2 + h], sem)
        return carry

    lax.fori_loop(lo, hi, _slab, 0)
    for h in range(2):
        pltpu.make_async_copy(out_hbm.at[0], stags[h], sems[h]).wait()


_gather_rows = functools.partial(
    pl.kernel,
    out_type=jax.ShapeDtypeStruct((L * 2, 2 * UNIT), jnp.float32),
    mesh=plsc.VectorSubcoreMesh(core_axis_name="c", subcore_axis_name="s"),
    scratch_types=[
        pltpu.VMEM((N_ROWS * D,), jnp.float32),
        pltpu.VMEM((1024,), jnp.int32),
        pltpu.VMEM((256,), jnp.int32),
        pltpu.VMEM((2 * UNIT,), jnp.float32),
        pltpu.VMEM((2 * UNIT,), jnp.float32),
        pltpu.SemaphoreType.DMA,
        pltpu.SemaphoreType.DMA,
        pltpu.SemaphoreType.DMA,
    ],
    compiler_params=pltpu.CompilerParams(
        use_tc_tiling_on_sc=False, needs_layout_passes=False
    ),
)(_sc_body)


def kernel(atom_indices, valence_configs, core_configs, W_valence, W_core):
    table = _build_table(valence_configs, core_configs, W_valence, W_core)
    idx4 = (
        atom_indices.astype(jnp.int32)
        .reshape(8, 128, L // 8, 8)
        .transpose(2, 0, 3, 1)
    )
    out = _gather_rows(table.reshape(N_ROWS * D), idx4)
    return (
        out.reshape(L, 8, 8, 8, 128)
        .transpose(2, 4, 0, 1, 3)
        .reshape(B, L, D)
    )


# speedup vs baseline: 25.0472x; 1.0921x over previous
import functools

import jax
import jax.numpy as jnp
from jax import lax
from jax.experimental import pallas as pl
from jax.experimental.pallas import tpu as pltpu
from jax.experimental.pallas import tpu_sc as plsc

K = 12
D = 64
N_ROWS = 19
B = 1024
L = 200

NC, NS = 2, 16
NW = NC * NS
L_UNITS_MAX = (L + NW - 1) // NW + 1
UNIT = 2 * 8 * 8 * 128
SLAB = D * B


def _table_body(vc_ref, cc_ref, wv_ref, wc_ref, out_ref):
    zero = jnp.zeros((1, K), jnp.float32)
    aug_v = jnp.concatenate([zero, vc_ref[...]], axis=0)
    aug_c = jnp.concatenate([zero, cc_ref[...]], axis=0)
    tv = jnp.dot(aug_v, wv_ref[...], preferred_element_type=jnp.float32,
                 precision=jax.lax.Precision.HIGHEST)
    tc = jnp.dot(aug_c, wc_ref[...], preferred_element_type=jnp.float32,
                 precision=jax.lax.Precision.HIGHEST)
    out_ref[...] = jnp.concatenate([tv, tc], axis=-1)


def _build_table(vc, cc, wv, wc):
    return pl.pallas_call(
        _table_body,
        out_shape=jax.ShapeDtypeStruct((N_ROWS, D), jnp.float32),
    )(vc, cc, wv, wc)


def _sc_body(table_hbm, idx_hbm, out_hbm,
             table_v, icol, svtab, stag_a, stag_b, sem_a, sem_b, isem):
    w = lax.axis_index("s") * NC + lax.axis_index("c")
    pltpu.sync_copy(table_hbm, table_v)

    iota = lax.iota(jnp.int32, 16)
    for d in range(16):
        cr = (iota + d) & 15
        svtab[pl.ds(d * 16, 16)] = (cr >> 3) * 8192 + (cr & 7) * 128 + iota

    lo = (w * L) // NW
    hi = ((w + 1) * L) // NW
    stags = (stag_a, stag_b)
    sems = (sem_a, sem_b)

    for bb in range(8):
        pltpu.async_copy(
            idx_hbm.at[lo >> 3, bb, lo & 7], icol.at[0, pl.ds(bb * 128, 128)],
            isem,
        )

    def _slab(l, carry):
        isel = (l - lo) & 1
        for bb in range(8):
            pltpu.make_async_copy(
                idx_hbm.at[0, 0, 0], icol.at[isel, pl.ds(bb * 128, 128)], isem
            ).wait()

        @pl.when(l + 1 < hi)
        def _prefetch():
            nxt = l + 1
            for bb in range(8):
                pltpu.async_copy(
                    idx_hbm.at[nxt >> 3, bb, nxt & 7],
                    icol.at[1 - isel, pl.ds(bb * 128, 128)],
                    isem,
                )

        for h in range(2):
            stag = stags[h]
            sem = sems[h]

            @pl.when(l > lo)
            def _drain(_stag=stag, _sem=sem):
                pltpu.make_async_copy(out_hbm.at[0], _stag, _sem).wait()

            @plsc.parallel_loop(0, 128, unroll=2)
            def _group(g2, _stag=stag, _h=h, _isel=isel):
                cqr = g2 >> 6
                g = g2 & 63
                v = icol[_isel, pl.ds(g * 16, 16)]
                v64 = v * D
                goff = cqr * UNIT + (g >> 3) * 1024 + (g & 7) * 16
                tb = _h * 32 + cqr * 16
                dg = iota
                for d in range(16):
                    vals = plsc.load_gather(
                        table_v.at[pl.ds(tb, (N_ROWS - 1) * D + 16)],
                        [v64 + dg],
                    )
                    sv = svtab[pl.ds(d * 16, 16)]
                    plsc.store_scatter(
                        _stag.at[pl.ds(goff, 8192 + 896 + 16)], [sv], vals
                    )
                    dg = (dg + 1) & 15

            pltpu.async_copy(stag, out_hbm.at[l * 2 + h], sem)
        return carry

    lax.fori_loop(lo, hi, _slab, 0)
    for h in range(2):
        pltpu.make_async_copy(out_hbm.at[0], stags[h], sems[h]).wait()


_gather_rows = functools.partial(
    pl.kernel,
    out_type=jax.ShapeDtypeStruct((L * 2, 2 * UNIT), jnp.float32),
    mesh=plsc.VectorSubcoreMesh(core_axis_name="c", subcore_axis_name="s"),
    scratch_types=[
        pltpu.VMEM((N_ROWS * D,), jnp.float32),
        pltpu.VMEM((2, 1024), jnp.int32),
        pltpu.VMEM((256,), jnp.int32),
        pltpu.VMEM((2 * UNIT,), jnp.float32),
        pltpu.VMEM((2 * UNIT,), jnp.float32),
        pltpu.SemaphoreType.DMA,
        pltpu.SemaphoreType.DMA,
        pltpu.SemaphoreType.DMA,
    ],
    compiler_params=pltpu.CompilerParams(
        use_tc_tiling_on_sc=False, needs_layout_passes=False
    ),
)(_sc_body)


def kernel(atom_indices, valence_configs, core_configs, W_valence, W_core):
    table = _build_table(valence_configs, core_configs, W_valence, W_core)
    idx4 = (
        atom_indices.astype(jnp.int32)
        .reshape(8, 128, L // 8, 8)
        .transpose(2, 0, 3, 1)
    )
    out = _gather_rows(table.reshape(N_ROWS * D), idx4)
    return (
        out.reshape(L, 8, 8, 8, 128)
        .transpose(2, 4, 0, 1, 3)
        .reshape(B, L, D)
    )


# speedup vs baseline: 26.0124x; 1.0385x over previous
import functools

import jax
import jax.numpy as jnp
from jax import lax
from jax.experimental import pallas as pl
from jax.experimental.pallas import tpu as pltpu
from jax.experimental.pallas import tpu_sc as plsc

K = 12
D = 64
N_ROWS = 19
B = 1024
L = 200

NC, NS = 2, 16
NW = NC * NS
L_UNITS_MAX = (L + NW - 1) // NW + 1
UNIT = 2 * 8 * 8 * 128
SLAB = D * B


def _table_body(vc_ref, cc_ref, wv_ref, wc_ref, out_ref):
    zero = jnp.zeros((1, K), jnp.float32)
    aug_v = jnp.concatenate([zero, vc_ref[...]], axis=0)
    aug_c = jnp.concatenate([zero, cc_ref[...]], axis=0)
    tv = jnp.dot(aug_v, wv_ref[...], preferred_element_type=jnp.float32,
                 precision=jax.lax.Precision.HIGHEST)
    tc = jnp.dot(aug_c, wc_ref[...], preferred_element_type=jnp.float32,
                 precision=jax.lax.Precision.HIGHEST)
    out_ref[...] = jnp.concatenate([tv, tc], axis=-1)


def _build_table(vc, cc, wv, wc):
    return pl.pallas_call(
        _table_body,
        out_shape=jax.ShapeDtypeStruct((N_ROWS, D), jnp.float32),
    )(vc, cc, wv, wc)


def _sc_body(table_hbm, idx_hbm, out_hbm,
             table_v, icol, svtab, stag_a, stag_b, sem_a, sem_b, isem):
    w = lax.axis_index("s") * NC + lax.axis_index("c")
    pltpu.sync_copy(table_hbm, table_v)

    iota = lax.iota(jnp.int32, 16)
    for d in range(16):
        cr = (iota + d) & 15
        svtab[pl.ds(d * 16, 16)] = (cr >> 3) * 8192 + (cr & 7) * 128 + iota

    lo = w * 25
    hi = lo + 25
    stags = (stag_a, stag_b)
    sems = (sem_a, sem_b)

    l0 = lo >> 2
    for bb in range(8):
        pltpu.async_copy(
            idx_hbm.at[l0 >> 3, bb, l0 & 7],
            icol.at[l0 & 1, pl.ds(bb * 128, 128)],
            isem,
        )

    def _unit(u, carry):
        l = u >> 2
        cq = u & 3
        isel = l & 1

        @pl.when((cq == 0) | (u == lo))
        def _drain_icol():
            for bb in range(8):
                pltpu.make_async_copy(
                    idx_hbm.at[0, 0, 0],
                    icol.at[isel, pl.ds(bb * 128, 128)],
                    isem,
                ).wait()

        @pl.when(((cq == 3) | (u == hi - 1)) & (u + 1 < hi))
        def _prefetch():
            nxt = l + 1
            for bb in range(8):
                pltpu.async_copy(
                    idx_hbm.at[nxt >> 3, bb, nxt & 7],
                    icol.at[1 - isel, pl.ds(bb * 128, 128)],
                    isem,
                )

        for par in range(2):
            stag = stags[par]
            sem = sems[par]

            @pl.when((u & 1) == par)
            def _do(_stag=stag, _sem=sem, _par=par):
                @pl.when(u >= lo + 2)
                def _drain():
                    pltpu.make_async_copy(out_hbm.at[0], _stag, _sem).wait()

                @plsc.parallel_loop(0, 64, unroll=2)
                def _group(g, _stag=_stag):
                    v = icol[isel, pl.ds(g * 16, 16)]
                    v64 = v * D
                    goff = (g >> 3) * 1024 + (g & 7) * 16
                    tb = cq * 16
                    dg = iota
                    for d in range(16):
                        vals = plsc.load_gather(
                            table_v.at[pl.ds(tb, (N_ROWS - 1) * D + 16)],
                            [v64 + dg],
                        )
                        sv = svtab[pl.ds(d * 16, 16)]
                        plsc.store_scatter(
                            _stag.at[pl.ds(goff, 8192 + 896 + 16)], [sv], vals
                        )
                        dg = (dg + 1) & 15

                pltpu.async_copy(_stag, out_hbm.at[u], _sem)
        return carry

    lax.fori_loop(lo, hi, _unit, 0)
    for par in range(2):
        pltpu.make_async_copy(out_hbm.at[0], stags[par], sems[par]).wait()


_gather_rows = functools.partial(
    pl.kernel,
    out_type=jax.ShapeDtypeStruct((L * 4, UNIT), jnp.float32),
    mesh=plsc.VectorSubcoreMesh(core_axis_name="c", subcore_axis_name="s"),
    scratch_types=[
        pltpu.VMEM((N_ROWS * D,), jnp.float32),
        pltpu.VMEM((2, 1024), jnp.int32),
        pltpu.VMEM((256,), jnp.int32),
        pltpu.VMEM((UNIT,), jnp.float32),
        pltpu.VMEM((UNIT,), jnp.float32),
        pltpu.SemaphoreType.DMA,
        pltpu.SemaphoreType.DMA,
        pltpu.SemaphoreType.DMA,
    ],
    compiler_params=pltpu.CompilerParams(
        use_tc_tiling_on_sc=False, needs_layout_passes=False
    ),
)(_sc_body)


def kernel(atom_indices, valence_configs, core_configs, W_valence, W_core):
    table = _build_table(valence_configs, core_configs, W_valence, W_core)
    idx4 = (
        atom_indices.astype(jnp.int32)
        .reshape(8, 128, L // 8, 8)
        .transpose(2, 0, 3, 1)
    )
    out = _gather_rows(table.reshape(N_ROWS * D), idx4)
    return (
        out.reshape(L, 8, 8, 8, 128)
        .transpose(2, 4, 0, 1, 3)
        .reshape(B, L, D)
    )
